# Initial kernel scaffold; baseline (speedup 1.0000x reference)
import functools

import jax
import jax.numpy as jnp
from jax import lax
from jax.experimental import pallas as pl
from jax.experimental.pallas import tpu as pltpu
from jax.experimental.pallas import tpu_sc as plsc

_NC = 2
_NS = 16
_NW = _NC * _NS

_BATCH = 128
_ROW_BLK = 1280


def _matmul_kernel(x_ref, w_ref, o_ref):
    o_ref[...] = jnp.dot(x_ref[...], w_ref[...],
                         preferred_element_type=jnp.float32)


def _sum2_matmul_kernel(a_ref, w_ref, o_ref):
    s = a_ref[0] + a_ref[1]
    o_ref[...] = jnp.dot(s, w_ref[...], preferred_element_type=jnp.float32)


def _make_sc_agg(n_pad, aug, nb):
    rows_per_tile = n_pad // _NS
    mesh = plsc.VectorSubcoreMesh(core_axis_name="c", subcore_axis_name="s",
                                  num_cores=_NC, num_subcores=_NS)

    @functools.partial(
        pl.kernel,
        out_type=jax.ShapeDtypeStruct((_NC, n_pad, aug), jnp.float32),
        mesh=mesh,
        scratch_types=[
            pltpu.VMEM((nb, _BATCH), jnp.int32),
            pltpu.VMEM((nb, _BATCH), jnp.int32),
            pltpu.VMEM((_BATCH, aug), jnp.float32),
            pltpu.VMEM_SHARED((n_pad, aug), jnp.float32),
            pltpu.SemaphoreType.DMA,
        ],
    )
    def sc_agg(nfeat_hbm, src_hbm, dst_hbm, zeros_hbm, out_hbm,
               src_v, dst_v, buf, acc, sem):
        cid = lax.axis_index("c")
        sid = lax.axis_index("s")
        wid = cid * _NS + sid
        r0 = sid * rows_per_tile

        pltpu.sync_copy(zeros_hbm, acc.at[pl.ds(r0, rows_per_tile)])
        pltpu.sync_copy(src_hbm.at[wid], src_v)
        pltpu.sync_copy(dst_hbm.at[wid], dst_v)
        plsc.subcore_barrier()

        def body(j, carry):
            pltpu.async_copy(nfeat_hbm.at[src_v.at[j]], buf, sem).wait()
            pltpu.sync_copy(buf, acc.at[dst_v.at[j]], add=True)
            return carry

        lax.fori_loop(0, nb, body, 0)

        plsc.subcore_barrier()
        pltpu.sync_copy(acc.at[pl.ds(r0, rows_per_tile)],
                        out_hbm.at[cid, pl.ds(r0, rows_per_tile)])

    return sc_agg


def kernel(x, edge_index, b_feat, W_feat, W_n, b_n):
    n, f = x.shape
    h = W_n.shape[0]
    e = edge_index.shape[1]
    aug = h + 16
    n_pad = 10240

    nb = -(-e // (_NW * _BATCH))
    e_pad = _NW * nb * _BATCH

    x_aug = jnp.zeros((n_pad, aug), jnp.float32)
    x_aug = x_aug.at[:n, :f].set(x).at[:n, f].set(1.0)

    w_big = jnp.zeros((aug, aug), jnp.float32)
    w_big = w_big.at[:f, :h].set(W_feat.T)
    w_big = w_big.at[f, :h].set((b_feat @ W_feat.T)[0])
    w_big = w_big.at[f, h].set(1.0)

    w_out = jnp.zeros((aug, h), jnp.float32)
    w_out = w_out.at[:h, :].set(W_n.T).at[h, :].set(b_n)

    src = jnp.pad(edge_index[0].astype(jnp.int32), (0, e_pad - e))
    dst = jnp.pad(edge_index[1].astype(jnp.int32), (0, e_pad - e),
                  constant_values=n)
    src_r = src.reshape(_NW, nb, _BATCH)
    dst_r = dst.reshape(_NW, nb, _BATCH)
    zeros = jnp.zeros((n_pad // _NS, aug), jnp.float32)

    grid = n_pad // _ROW_BLK
    nfeat_aug = pl.pallas_call(
        _matmul_kernel,
        grid=(grid,),
        in_specs=[pl.BlockSpec((_ROW_BLK, aug), lambda i: (i, 0)),
                  pl.BlockSpec((aug, aug), lambda i: (0, 0))],
        out_specs=pl.BlockSpec((_ROW_BLK, aug), lambda i: (i, 0)),
        out_shape=jax.ShapeDtypeStruct((n_pad, aug), jnp.float32),
    )(x_aug, w_big)

    acc = _make_sc_agg(n_pad, aug, nb)(nfeat_aug, src_r, dst_r, zeros)

    out = pl.pallas_call(
        _sum2_matmul_kernel,
        grid=(grid,),
        in_specs=[pl.BlockSpec((_NC, _ROW_BLK, aug), lambda i: (0, i, 0)),
                  pl.BlockSpec((aug, h), lambda i: (0, 0))],
        out_specs=pl.BlockSpec((_ROW_BLK, h), lambda i: (i, 0)),
        out_shape=jax.ShapeDtypeStruct((n_pad, h), jnp.float32),
    )(acc, w_out)

    return out[:n]


# speedup vs baseline: 4.5146x; 4.5146x over previous
import functools

import jax
import jax.numpy as jnp
from jax import lax
from jax.experimental import pallas as pl
from jax.experimental.pallas import tpu as pltpu
from jax.experimental.pallas import tpu_sc as plsc

_NC = 2
_NS = 16
_NW = _NC * _NS

_BATCH = 128
_ROW_BLK = 1280


def _node_kernel(x_ref, bf_ref, w1_ref, w2_ref, bn_ref, o_ref):
    a = x_ref[...] + bf_ref[...]
    dn = (((1,), (1,)), ((), ()))
    nf = lax.dot_general(a, w1_ref[...], dn,
                         preferred_element_type=jnp.float32)
    o_ref[...] = lax.dot_general(nf, w2_ref[...], dn,
                                 preferred_element_type=jnp.float32) + bn_ref[...]


def _sum2_kernel(a_ref, o_ref):
    o_ref[...] = a_ref[0] + a_ref[1]


def _make_sc_agg(n_pad, h, nb):
    rows_per_tile = n_pad // _NS
    mesh = plsc.VectorSubcoreMesh(core_axis_name="c", subcore_axis_name="s",
                                  num_cores=_NC, num_subcores=_NS)

    @functools.partial(
        pl.kernel,
        out_type=jax.ShapeDtypeStruct((_NC, n_pad, h), jnp.float32),
        mesh=mesh,
        scratch_types=[
            pltpu.VMEM((nb, _BATCH), jnp.int32),
            pltpu.VMEM((nb, _BATCH), jnp.int32),
            pltpu.VMEM((_BATCH, h), jnp.float32),
            pltpu.VMEM_SHARED((n_pad, h), jnp.float32),
            pltpu.SemaphoreType.DMA,
        ],
    )
    def sc_agg(t_hbm, src_hbm, dst_hbm, zeros_hbm, out_hbm,
               src_v, dst_v, buf, acc, sem):
        cid = lax.axis_index("c")
        sid = lax.axis_index("s")
        wid = cid * _NS + sid
        r0 = sid * rows_per_tile

        pltpu.sync_copy(zeros_hbm, acc.at[pl.ds(r0, rows_per_tile)])
        pltpu.sync_copy(src_hbm.at[wid], src_v)
        pltpu.sync_copy(dst_hbm.at[wid], dst_v)
        plsc.subcore_barrier()

        def body(j, carry):
            pltpu.async_copy(t_hbm.at[src_v.at[j]], buf, sem).wait()
            pltpu.sync_copy(buf, acc.at[dst_v.at[j]], add=True)
            return carry

        lax.fori_loop(0, nb, body, 0)

        plsc.subcore_barrier()
        pltpu.sync_copy(acc.at[pl.ds(r0, rows_per_tile)],
                        out_hbm.at[cid, pl.ds(r0, rows_per_tile)])

    return sc_agg


def kernel(x, edge_index, b_feat, W_feat, W_n, b_n):
    n, f = x.shape
    h = W_n.shape[0]
    e = edge_index.shape[1]
    n_pad = 10240

    nb = -(-e // (_NW * _BATCH))
    e_pad = _NW * nb * _BATCH

    x_pad = jnp.zeros((n_pad, f), jnp.float32).at[:n].set(x)
    src = jnp.pad(edge_index[0].astype(jnp.int32), (0, e_pad - e))
    dst = jnp.pad(edge_index[1].astype(jnp.int32), (0, e_pad - e),
                  constant_values=n)
    src_r = src.reshape(_NW, nb, _BATCH)
    dst_r = dst.reshape(_NW, nb, _BATCH)
    zeros = jnp.zeros((n_pad // _NS, h), jnp.float32)
    bn_row = b_n.reshape(1, h)

    grid = n_pad // _ROW_BLK
    t = pl.pallas_call(
        _node_kernel,
        grid=(grid,),
        in_specs=[pl.BlockSpec((_ROW_BLK, f), lambda i: (i, 0)),
                  pl.BlockSpec((1, f), lambda i: (0, 0)),
                  pl.BlockSpec((h, f), lambda i: (0, 0)),
                  pl.BlockSpec((h, h), lambda i: (0, 0)),
                  pl.BlockSpec((1, h), lambda i: (0, 0))],
        out_specs=pl.BlockSpec((_ROW_BLK, h), lambda i: (i, 0)),
        out_shape=jax.ShapeDtypeStruct((n_pad, h), jnp.float32),
    )(x_pad, b_feat, W_feat, W_n, bn_row)

    acc = _make_sc_agg(n_pad, h, nb)(t, src_r, dst_r, zeros)

    out = pl.pallas_call(
        _sum2_kernel,
        grid=(grid,),
        in_specs=[pl.BlockSpec((_NC, _ROW_BLK, h), lambda i: (0, i, 0))],
        out_specs=pl.BlockSpec((_ROW_BLK, h), lambda i: (i, 0)),
        out_shape=jax.ShapeDtypeStruct((n_pad, h), jnp.float32),
    )(acc)

    return out[:n]
